# baseline (speedup 1.0000x reference)
import jax
import jax.numpy as jnp
from jax import lax
from jax.experimental import pallas as pl
from jax.experimental.pallas import tpu as pltpu

_H = 768
_HG = 384
_E = 64
_K = 12
_N = 32768
_T = 4096

_PREC = lax.Precision.DEFAULT


def _router_tile(x_ref, w1_ref, b1_ref, w2_ref, b2_ref, rw_ref, sm_ref,
                 scaled_ref, idx_ref, ew_ref, var_ref, ent_ref,
                 load_acc, ent_acc):
    i = pl.program_id(0)
    nb = pl.num_programs(0)

    @pl.when(i == 0)
    def _init():
        load_acc[...] = jnp.zeros_like(load_acc)
        ent_acc[0] = jnp.float32(0.0)

    x = x_ref[...]
    h1t = lax.dot_general(w1_ref[...], x, (((1,), (1,)), ((), ())),
                          precision=_PREC, preferred_element_type=jnp.float32)
    h1t = jnp.maximum(h1t + b1_ref[...], 0.0)
    gt = lax.dot_general(w2_ref[...], h1t, (((1,), (0,)), ((), ())),
                         precision=_PREC, preferred_element_type=jnp.float32)
    gt = jax.nn.sigmoid(gt + b2_ref[...])
    mw = rw_ref[...] * jax.nn.sigmoid(sm_ref[...])
    logits_t = lax.dot_general(mw, x, (((1,), (1,)), ((), ())),
                               precision=_PREC, preferred_element_type=jnp.float32)
    st = logits_t * gt
    scaled_ref[...] = st.T

    bits = lax.bitcast_convert_type(st, jnp.int32)
    key = bits ^ (jnp.right_shift(bits, 31) & jnp.int32(0x7FFFFFFF))
    inv_iota = jnp.int32(_E - 1) - lax.broadcasted_iota(jnp.int32, st.shape, 0)
    work = ((key + jnp.int32(32)) & jnp.int32(~63)) | inv_iota
    kmax = []
    for _ in range(_K):
        m = jnp.max(work, axis=0, keepdims=True)
        kmax.append(m)
        work = jnp.where(work == m, jnp.int32(-0x80000000), work)

    km = jnp.concatenate(kmax, axis=0)
    idx_ref[...] = jnp.int32(_E - 1) - (km & jnp.int32(63))
    vbits = km ^ (jnp.right_shift(km, 31) & jnp.int32(0x7FFFFFFF))
    v = lax.bitcast_convert_type(vbits, jnp.float32)
    exps = jnp.exp(v - v[0:1])
    ew_ref[...] = exps / jnp.sum(exps, axis=0, keepdims=True)

    m0 = v[0:1]
    d = st - m0
    q = jnp.exp(d)
    z = jnp.sum(q, axis=0, keepdims=True)
    load_acc[...] += jnp.sum(q / z, axis=1, keepdims=True)
    ent_tok = jnp.log(z) - jnp.sum(q * d, axis=0, keepdims=True) / z
    ent_acc[0] += jnp.sum(ent_tok)

    @pl.when(i == nb - 1)
    def _finalize():
        load = load_acc[...] / jnp.float32(_N)
        mu = jnp.mean(load)
        var_ref[...] = jnp.mean((load - mu) ** 2, keepdims=True).reshape(1, 1)
        ent_ref[...] = (ent_acc[0] / jnp.float32(_N)).reshape(1, 1)


def kernel(hidden_states, router_weight, sparsity_mask, gate_w1, gate_b1,
           gate_w2, gate_b2):
    b1 = gate_b1.reshape(_HG, 1)
    b2 = gate_b2.reshape(_E, 1)

    grid = (_N // _T,)
    out_shape = (
        jax.ShapeDtypeStruct((_N, _E), jnp.float32),
        jax.ShapeDtypeStruct((_K, _N), jnp.int32),
        jax.ShapeDtypeStruct((_K, _N), jnp.float32),
        jax.ShapeDtypeStruct((1, 1), jnp.float32),
        jax.ShapeDtypeStruct((1, 1), jnp.float32),
    )
    in_specs = [
        pl.BlockSpec((_T, _H), lambda i: (i, 0)),
        pl.BlockSpec((_HG, _H), lambda i: (0, 0)),
        pl.BlockSpec((_HG, 1), lambda i: (0, 0)),
        pl.BlockSpec((_E, _HG), lambda i: (0, 0)),
        pl.BlockSpec((_E, 1), lambda i: (0, 0)),
        pl.BlockSpec((_E, _H), lambda i: (0, 0)),
        pl.BlockSpec((_E, _H), lambda i: (0, 0)),
    ]
    out_specs = (
        pl.BlockSpec((_T, _E), lambda i: (i, 0)),
        pl.BlockSpec((_K, _T), lambda i: (0, i)),
        pl.BlockSpec((_K, _T), lambda i: (0, i)),
        pl.BlockSpec((1, 1), lambda i: (0, 0)),
        pl.BlockSpec((1, 1), lambda i: (0, 0)),
    )
    scaled_t, idx_t, ew_t, var, ent = pl.pallas_call(
        _router_tile,
        grid=grid,
        in_specs=in_specs,
        out_specs=out_specs,
        out_shape=out_shape,
        scratch_shapes=[
            pltpu.VMEM((_E, 1), jnp.float32),
            pltpu.SMEM((1,), jnp.float32),
        ],
        compiler_params=pltpu.CompilerParams(
            dimension_semantics=("arbitrary",),
        ),
    )(hidden_states, gate_w1, b1, gate_w2, b2, router_weight, sparsity_mask)
    return (scaled_t, idx_t.T, ew_t.T, var[0, 0], ent[0, 0])


# speedup vs baseline: 1.2461x; 1.2461x over previous
import jax
import jax.numpy as jnp
from jax import lax
from jax.experimental import pallas as pl
from jax.experimental.pallas import tpu as pltpu

_H = 768
_HG = 384
_E = 64
_K = 12
_N = 32768
_T = 4096

_PREC = lax.Precision.DEFAULT


def _router_tile(x_ref, w1_ref, b1_ref, w2_ref, b2_ref, rw_ref, sm_ref,
                 scaled_ref, idx_ref, ew_ref, var_ref, ent_ref,
                 load_acc, ent_acc):
    i = pl.program_id(0)
    nb = pl.num_programs(0)

    @pl.when(i == 0)
    def _init():
        load_acc[...] = jnp.zeros_like(load_acc)
        ent_acc[0] = jnp.float32(0.0)

    x = x_ref[...]
    mw = rw_ref[...] * jax.nn.sigmoid(sm_ref[...])
    wcat = jnp.concatenate([w1_ref[...], mw], axis=0)
    cat_t = lax.dot_general(wcat, x, (((1,), (1,)), ((), ())),
                            precision=_PREC, preferred_element_type=jnp.float32)
    h1t = jnp.maximum(cat_t[:_HG] + b1_ref[...], 0.0)
    logits_t = cat_t[_HG:]
    gt = lax.dot_general(w2_ref[...], h1t, (((1,), (0,)), ((), ())),
                         precision=_PREC, preferred_element_type=jnp.float32)
    gt = jax.nn.sigmoid(gt + b2_ref[...])
    st = logits_t * gt
    scaled_ref[...] = st

    bits = lax.bitcast_convert_type(st, jnp.int32)
    key = bits ^ (jnp.right_shift(bits, 31) & jnp.int32(0x7FFFFFFF))
    inv_iota = jnp.int32(_E - 1) - lax.broadcasted_iota(jnp.int32, st.shape, 0)
    work = ((key + jnp.int32(32)) & jnp.int32(~63)) | inv_iota
    kmax = []
    for _ in range(_K):
        m = jnp.max(work, axis=0, keepdims=True)
        kmax.append(m)
        work = jnp.where(work == m, jnp.int32(-0x80000000), work)

    km = jnp.concatenate(kmax, axis=0)
    idx_ref[...] = jnp.int32(_E - 1) - (km & jnp.int32(63))
    vbits = km ^ (jnp.right_shift(km, 31) & jnp.int32(0x7FFFFFFF))
    v = lax.bitcast_convert_type(vbits, jnp.float32)
    exps = jnp.exp(v - v[0:1])
    ew_ref[...] = exps / jnp.sum(exps, axis=0, keepdims=True)

    m0 = v[0:1]
    d = st - m0
    q = jnp.exp(d)
    z = jnp.sum(q, axis=0, keepdims=True)
    load_acc[...] += jnp.sum(q / z, axis=1, keepdims=True)
    ent_tok = jnp.log(z) - jnp.sum(q * d, axis=0, keepdims=True) / z
    ent_acc[0] += jnp.sum(ent_tok)

    @pl.when(i == nb - 1)
    def _finalize():
        load = load_acc[...] / jnp.float32(_N)
        mu = jnp.mean(load)
        var_ref[...] = jnp.mean((load - mu) ** 2, keepdims=True).reshape(1, 1)
        ent_ref[...] = (ent_acc[0] / jnp.float32(_N)).reshape(1, 1)


def kernel(hidden_states, router_weight, sparsity_mask, gate_w1, gate_b1,
           gate_w2, gate_b2):
    b1 = gate_b1.reshape(_HG, 1)
    b2 = gate_b2.reshape(_E, 1)

    grid = (_N // _T,)
    out_shape = (
        jax.ShapeDtypeStruct((_E, _N), jnp.float32),
        jax.ShapeDtypeStruct((_K, _N), jnp.int32),
        jax.ShapeDtypeStruct((_K, _N), jnp.float32),
        jax.ShapeDtypeStruct((1, 1), jnp.float32),
        jax.ShapeDtypeStruct((1, 1), jnp.float32),
    )
    in_specs = [
        pl.BlockSpec((_T, _H), lambda i: (i, 0)),
        pl.BlockSpec((_HG, _H), lambda i: (0, 0)),
        pl.BlockSpec((_HG, 1), lambda i: (0, 0)),
        pl.BlockSpec((_E, _HG), lambda i: (0, 0)),
        pl.BlockSpec((_E, 1), lambda i: (0, 0)),
        pl.BlockSpec((_E, _H), lambda i: (0, 0)),
        pl.BlockSpec((_E, _H), lambda i: (0, 0)),
    ]
    out_specs = (
        pl.BlockSpec((_E, _T), lambda i: (0, i)),
        pl.BlockSpec((_K, _T), lambda i: (0, i)),
        pl.BlockSpec((_K, _T), lambda i: (0, i)),
        pl.BlockSpec((1, 1), lambda i: (0, 0)),
        pl.BlockSpec((1, 1), lambda i: (0, 0)),
    )
    scaled_t, idx_t, ew_t, var, ent = pl.pallas_call(
        _router_tile,
        grid=grid,
        in_specs=in_specs,
        out_specs=out_specs,
        out_shape=out_shape,
        scratch_shapes=[
            pltpu.VMEM((_E, 1), jnp.float32),
            pltpu.SMEM((1,), jnp.float32),
        ],
        compiler_params=pltpu.CompilerParams(
            dimension_semantics=("arbitrary",),
        ),
    )(hidden_states, gate_w1, b1, gate_w2, b2, router_weight, sparsity_mask)
    return (scaled_t.T, idx_t.T, ew_t.T, var[0, 0], ent[0, 0])


# speedup vs baseline: 1.2462x; 1.0000x over previous
import jax
import jax.numpy as jnp
from jax import lax
from jax.experimental import pallas as pl
from jax.experimental.pallas import tpu as pltpu

_H = 768
_HG = 384
_E = 64
_K = 12
_N = 32768
_T = 4096

_PREC = lax.Precision.DEFAULT


def _router_tile(x_ref, w1_ref, b1_ref, w2_ref, b2_ref, rw_ref, sm_ref,
                 scaled_ref, idx_ref, ew_ref, var_ref, ent_ref,
                 wcat_s, load_acc, ent_acc):
    i = pl.program_id(0)
    nb = pl.num_programs(0)

    @pl.when(i == 0)
    def _init():
        load_acc[...] = jnp.zeros_like(load_acc)
        ent_acc[0] = jnp.float32(0.0)
        mw = rw_ref[...] * jax.nn.sigmoid(sm_ref[...])
        wcat_s[...] = jnp.concatenate([w1_ref[...], mw], axis=0)

    x = x_ref[...]
    cat_t = lax.dot_general(wcat_s[...], x, (((1,), (1,)), ((), ())),
                            precision=_PREC, preferred_element_type=jnp.float32)
    h1t = jnp.maximum(cat_t[:_HG] + b1_ref[...], 0.0)
    logits_t = cat_t[_HG:]
    gt = lax.dot_general(w2_ref[...], h1t, (((1,), (0,)), ((), ())),
                         precision=_PREC, preferred_element_type=jnp.float32)
    gt = jax.nn.sigmoid(gt + b2_ref[...])
    st = logits_t * gt
    scaled_ref[...] = st

    bits = lax.bitcast_convert_type(st, jnp.int32)
    key = bits ^ (jnp.right_shift(bits, 31) & jnp.int32(0x7FFFFFFF))
    inv_iota = jnp.int32(_E - 1) - lax.broadcasted_iota(jnp.int32, st.shape, 0)
    work = ((key + jnp.int32(32)) & jnp.int32(~63)) | inv_iota
    kmax = []
    for _ in range(_K):
        m = jnp.max(work, axis=0, keepdims=True)
        kmax.append(m)
        work = jnp.where(work == m, jnp.int32(-0x80000000), work)

    km = jnp.concatenate(kmax, axis=0)
    idx_ref[...] = jnp.int32(_E - 1) - (km & jnp.int32(63))
    vbits = km ^ (jnp.right_shift(km, 31) & jnp.int32(0x7FFFFFFF))
    v = lax.bitcast_convert_type(vbits, jnp.float32)
    exps = jnp.exp(v - v[0:1])
    ew_ref[...] = exps / jnp.sum(exps, axis=0, keepdims=True)

    m0 = v[0:1]
    d = st - m0
    q = jnp.exp(d)
    z = jnp.sum(q, axis=0, keepdims=True)
    load_acc[...] += jnp.sum(q / z, axis=1, keepdims=True)
    ent_tok = jnp.log(z) - jnp.sum(q * d, axis=0, keepdims=True) / z
    ent_acc[0] += jnp.sum(ent_tok)

    @pl.when(i == nb - 1)
    def _finalize():
        load = load_acc[...] / jnp.float32(_N)
        mu = jnp.mean(load)
        var_ref[...] = jnp.mean((load - mu) ** 2, keepdims=True).reshape(1, 1)
        ent_ref[...] = (ent_acc[0] / jnp.float32(_N)).reshape(1, 1)


def kernel(hidden_states, router_weight, sparsity_mask, gate_w1, gate_b1,
           gate_w2, gate_b2):
    b1 = gate_b1.reshape(_HG, 1)
    b2 = gate_b2.reshape(_E, 1)

    grid = (_N // _T,)
    out_shape = (
        jax.ShapeDtypeStruct((_E, _N), jnp.float32),
        jax.ShapeDtypeStruct((_K, _N), jnp.int32),
        jax.ShapeDtypeStruct((_K, _N), jnp.float32),
        jax.ShapeDtypeStruct((1, 1), jnp.float32),
        jax.ShapeDtypeStruct((1, 1), jnp.float32),
    )
    in_specs = [
        pl.BlockSpec((_T, _H), lambda i: (i, 0)),
        pl.BlockSpec((_HG, _H), lambda i: (0, 0)),
        pl.BlockSpec((_HG, 1), lambda i: (0, 0)),
        pl.BlockSpec((_E, _HG), lambda i: (0, 0)),
        pl.BlockSpec((_E, 1), lambda i: (0, 0)),
        pl.BlockSpec((_E, _H), lambda i: (0, 0)),
        pl.BlockSpec((_E, _H), lambda i: (0, 0)),
    ]
    out_specs = (
        pl.BlockSpec((_E, _T), lambda i: (0, i)),
        pl.BlockSpec((_K, _T), lambda i: (0, i)),
        pl.BlockSpec((_K, _T), lambda i: (0, i)),
        pl.BlockSpec((1, 1), lambda i: (0, 0)),
        pl.BlockSpec((1, 1), lambda i: (0, 0)),
    )
    scaled_t, idx_t, ew_t, var, ent = pl.pallas_call(
        _router_tile,
        grid=grid,
        in_specs=in_specs,
        out_specs=out_specs,
        out_shape=out_shape,
        scratch_shapes=[
            pltpu.VMEM((_HG + _E, _H), jnp.float32),
            pltpu.VMEM((_E, 1), jnp.float32),
            pltpu.SMEM((1,), jnp.float32),
        ],
        compiler_params=pltpu.CompilerParams(
            dimension_semantics=("arbitrary",),
        ),
    )(hidden_states, gate_w1, b1, gate_w2, b2, router_weight, sparsity_mask)
    return (scaled_t.T, idx_t.T, ew_t.T, var[0, 0], ent[0, 0])
